# baseline (speedup 1.0000x reference)
import functools
import jax
import jax.numpy as jnp
from jax import lax
from jax.experimental import pallas as pl
from jax.experimental.pallas import tpu as pltpu
from jax.experimental.pallas import tpu_sc as plsc

NUM_EXPERTS = 16
HIDDEN = 64
FLAT = NUM_EXPERTS * HIDDEN


def _pass1_body(x_ref, w1cat_ref, b1cat_ref, gw2t_ref, gb2_ref,
                h_ref, logits_ref):
    xb16 = x_ref[...].astype(jnp.bfloat16)
    acc = jnp.maximum(
        jnp.dot(xb16, w1cat_ref[...], preferred_element_type=jnp.float32)
        + b1cat_ref[...], 0.0)
    h_ref[...] = acc[:, :FLAT].astype(jnp.bfloat16)
    g1 = acc[:, FLAT:]
    logits_ref[...] = (
        jnp.dot(g1, gw2t_ref[...], preferred_element_type=jnp.float32)
        + gb2_ref[...])


def _pass2_body(h_ref, w_ref, w2_ref, eb2_ref, emat_ref, out_ref):
    w = w_ref[...]
    f = w.shape[0]
    c = jax.lax.dot_general(w.astype(jnp.bfloat16), h_ref[...],
                            (((0,), (0,)), ((), ())),
                            preferred_element_type=jnp.float32)
    s = jnp.sum(c * emat_ref[...], axis=0, keepdims=True)
    wsum = jnp.sum(w, axis=0, keepdims=True)
    out = (jnp.dot(s.astype(jnp.bfloat16), w2_ref[...],
                   preferred_element_type=jnp.float32)
           + jnp.dot(wsum, eb2_ref[...], preferred_element_type=jnp.float32))
    out_ref[...] = (out * (1.0 / f))[None]


def _make_sc_router(T, rows_per_worker):
    mesh = plsc.VectorSubcoreMesh(core_axis_name="c", subcore_axis_name="s")
    info = plsc.get_sparse_core_info()
    num_cores = info.num_cores

    @functools.partial(
        pl.kernel, mesh=mesh,
        compiler_params=pltpu.CompilerParams(needs_layout_passes=False),
        out_type=jax.ShapeDtypeStruct((T, NUM_EXPERTS), jnp.float32),
        scratch_types=[
            pltpu.VMEM((rows_per_worker, NUM_EXPERTS), jnp.float32),
            pltpu.VMEM((rows_per_worker, NUM_EXPERTS), jnp.float32),
        ],
    )
    def route(logits_hbm, w_hbm, lbuf, wbuf):
        wid = lax.axis_index("s") * num_cores + lax.axis_index("c")
        base = wid * rows_per_worker
        pltpu.sync_copy(logits_hbm.at[pl.ds(base, rows_per_worker)], lbuf)

        def body(i, carry):
            lv = lbuf[i]
            m = jnp.max(lv)
            el = jnp.exp(lv - m)
            z = jnp.sum(el)
            m1 = jnp.max(el)
            el2 = jnp.where(el == m1, -1.0, el)
            m2 = jnp.max(el2)
            wbuf[i] = jnp.where(el >= m2, el, 0.0) / z
            return carry

        lax.fori_loop(0, rows_per_worker, body, 0)
        pltpu.sync_copy(wbuf, w_hbm.at[pl.ds(base, rows_per_worker)])

    return route


def kernel(x, gw1, gb1, gw2, gb2, ew1, eb1, ew2, eb2):
    B, F, D = x.shape
    E, H, _ = ew1.shape
    O = ew2.shape[1]
    T = B * F

    xf = x.reshape(T, D)
    w1t = ew1.reshape(E * H, D).T.astype(jnp.bfloat16)
    gw1t = gw1.T.astype(jnp.bfloat16)
    w1cat = jnp.concatenate([w1t, gw1t], axis=1)
    b1cat = jnp.concatenate(
        [eb1.reshape(1, E * H), gb1.reshape(1, H)], axis=1)
    gw2t = gw2.T
    gb2r = gb2.reshape(1, E)
    w2 = ew2.transpose(0, 2, 1).reshape(E * H, O).astype(jnp.bfloat16)
    emat = jnp.kron(jnp.eye(E, dtype=x.dtype), jnp.ones((1, H), dtype=x.dtype))

    full = lambda *shape: pl.BlockSpec(shape, lambda b: (0,) * len(shape))

    info = plsc.get_sparse_core_info()
    num_workers = info.num_cores * info.num_subcores

    NH = 2
    BH = B // NH
    TH = BH * F
    router = _make_sc_router(TH, TH // num_workers)

    def pass1(xh):
        return pl.pallas_call(
            _pass1_body,
            grid=(BH,),
            in_specs=[
                pl.BlockSpec((F, D), lambda b: (b, 0)),
                full(D, E * H + H), full(1, E * H + H), full(H, E),
                full(1, E),
            ],
            out_specs=[
                pl.BlockSpec((F, E * H), lambda b: (b, 0)),
                pl.BlockSpec((F, E), lambda b: (b, 0)),
            ],
            out_shape=[
                jax.ShapeDtypeStruct((TH, E * H), jnp.bfloat16),
                jax.ShapeDtypeStruct((TH, E), jnp.float32),
            ],
        )(xh, w1cat, b1cat, gw2t, gb2r)

    def pass2(hh, wh):
        return pl.pallas_call(
            _pass2_body,
            grid=(BH,),
            in_specs=[
                pl.BlockSpec((F, E * H), lambda b: (b, 0)),
                pl.BlockSpec((F, E), lambda b: (b, 0)),
                full(E * H, O), full(E, O), full(E, E * H),
            ],
            out_specs=pl.BlockSpec((1, 1, O), lambda b: (b, 0, 0)),
            out_shape=jax.ShapeDtypeStruct((BH, 1, O), x.dtype),
        )(hh, wh, w2, eb2, emat)

    hs, ls, ws, outs = [], [], [], []
    for k in range(NH):
        hk, lk = pass1(xf[k * TH:(k + 1) * TH])
        hs.append(hk)
        ws.append(router(lk))
    for k in range(NH):
        outs.append(pass2(hs[k], ws[k]))
    out = jnp.concatenate(outs, axis=0)
    return out.reshape(B, 1, 1, O)


# speedup vs baseline: 1.5425x; 1.5425x over previous
import functools
import jax
import jax.numpy as jnp
from jax import lax
from jax.experimental import pallas as pl
from jax.experimental.pallas import tpu as pltpu
from jax.experimental.pallas import tpu_sc as plsc

NUM_EXPERTS = 16
HIDDEN = 64
FLAT = NUM_EXPERTS * HIDDEN


def _pass1_body(x_ref, w1cat_ref, b1cat_ref, gw2t_ref, gb2_ref,
                h_ref, logits_ref):
    xb16 = x_ref[...].astype(jnp.bfloat16)
    acc = jnp.maximum(
        jnp.dot(xb16, w1cat_ref[...], preferred_element_type=jnp.float32)
        + b1cat_ref[...], 0.0)
    h_ref[...] = acc[:, :FLAT].astype(jnp.float8_e4m3fn)
    g1 = acc[:, FLAT:]
    logits_ref[...] = (
        jnp.dot(g1, gw2t_ref[...], preferred_element_type=jnp.float32)
        + gb2_ref[...])


def _pass2_body(h_ref, w_ref, w2_ref, eb2_ref, emat_ref, out_ref):
    w = w_ref[...]
    f = w.shape[0]
    c = jax.lax.dot_general(w.astype(jnp.bfloat16),
                            h_ref[...].astype(jnp.bfloat16),
                            (((0,), (0,)), ((), ())),
                            preferred_element_type=jnp.float32)
    s = jnp.sum(c * emat_ref[...], axis=0, keepdims=True)
    wsum = jnp.sum(w, axis=0, keepdims=True)
    out = (jnp.dot(s.astype(jnp.bfloat16), w2_ref[...],
                   preferred_element_type=jnp.float32)
           + jnp.dot(wsum, eb2_ref[...], preferred_element_type=jnp.float32))
    out_ref[...] = (out * (1.0 / f))[None]


def _make_sc_router(T, rows_per_worker):
    mesh = plsc.VectorSubcoreMesh(core_axis_name="c", subcore_axis_name="s")
    info = plsc.get_sparse_core_info()
    num_cores = info.num_cores

    @functools.partial(
        pl.kernel, mesh=mesh,
        compiler_params=pltpu.CompilerParams(needs_layout_passes=False),
        out_type=jax.ShapeDtypeStruct((T, NUM_EXPERTS), jnp.float32),
        scratch_types=[
            pltpu.VMEM((rows_per_worker, NUM_EXPERTS), jnp.float32),
            pltpu.VMEM((rows_per_worker, NUM_EXPERTS), jnp.float32),
        ],
    )
    def route(logits_hbm, w_hbm, lbuf, wbuf):
        wid = lax.axis_index("s") * num_cores + lax.axis_index("c")
        base = wid * rows_per_worker
        pltpu.sync_copy(logits_hbm.at[pl.ds(base, rows_per_worker)], lbuf)

        def body(i, carry):
            lv = lbuf[i]
            m = jnp.max(lv)
            el = jnp.exp(lv - m)
            z = jnp.sum(el)
            m1 = jnp.max(el)
            el2 = jnp.where(el == m1, -1.0, el)
            m2 = jnp.max(el2)
            wbuf[i] = jnp.where(el >= m2, el, 0.0) / z
            return carry

        lax.fori_loop(0, rows_per_worker, body, 0)
        pltpu.sync_copy(wbuf, w_hbm.at[pl.ds(base, rows_per_worker)])

    return route


def kernel(x, gw1, gb1, gw2, gb2, ew1, eb1, ew2, eb2):
    B, F, D = x.shape
    E, H, _ = ew1.shape
    O = ew2.shape[1]
    T = B * F

    xf = x.reshape(T, D)
    w1t = ew1.reshape(E * H, D).T.astype(jnp.bfloat16)
    gw1t = gw1.T.astype(jnp.bfloat16)
    w1cat = jnp.concatenate([w1t, gw1t], axis=1)
    b1cat = jnp.concatenate(
        [eb1.reshape(1, E * H), gb1.reshape(1, H)], axis=1)
    gw2t = gw2.T
    gb2r = gb2.reshape(1, E)
    w2 = ew2.transpose(0, 2, 1).reshape(E * H, O).astype(jnp.bfloat16)
    emat = jnp.kron(jnp.eye(E, dtype=x.dtype), jnp.ones((1, H), dtype=x.dtype))

    full = lambda *shape: pl.BlockSpec(shape, lambda b: (0,) * len(shape))

    h, logits = pl.pallas_call(
        _pass1_body,
        grid=(B,),
        in_specs=[
            pl.BlockSpec((F, D), lambda b: (b, 0)),
            full(D, E * H + H), full(1, E * H + H), full(H, E), full(1, E),
        ],
        out_specs=[
            pl.BlockSpec((F, E * H), lambda b: (b, 0)),
            pl.BlockSpec((F, E), lambda b: (b, 0)),
        ],
        out_shape=[
            jax.ShapeDtypeStruct((T, E * H), jnp.float8_e4m3fn),
            jax.ShapeDtypeStruct((T, E), jnp.float32),
        ],
    )(xf, w1cat, b1cat, gw2t, gb2r)

    info = plsc.get_sparse_core_info()
    num_workers = info.num_cores * info.num_subcores
    w = _make_sc_router(T, T // num_workers)(logits)

    out = pl.pallas_call(
        _pass2_body,
        grid=(B,),
        in_specs=[
            pl.BlockSpec((F, E * H), lambda b: (b, 0)),
            pl.BlockSpec((F, E), lambda b: (b, 0)),
            full(E * H, O), full(E, O), full(E, E * H),
        ],
        out_specs=pl.BlockSpec((1, 1, O), lambda b: (b, 0, 0)),
        out_shape=jax.ShapeDtypeStruct((B, 1, O), x.dtype),
    )(h, w, w2, eb2, emat)
    return out.reshape(B, 1, 1, O)
